# baseline (speedup 1.0000x reference)
import functools
import math

import jax
import jax.numpy as jnp
from jax import lax
from jax.experimental import pallas as pl
from jax.experimental.pallas import tpu as pltpu
from jax.experimental.pallas import tpu_sc as plsc

D = 64
NW = 32
CHUNK = 128
SCALE = 1.0 / math.sqrt(D)


def _make_emb_kernel(b_total: int):
    b_per_w = b_total // NW
    n_chunks = b_per_w // CHUNK
    mesh = plsc.VectorSubcoreMesh(core_axis_name="c", subcore_axis_name="s")

    @functools.partial(
        pl.kernel,
        out_type=jax.ShapeDtypeStruct((b_total, D), jnp.float32),
        mesh=mesh,
        scratch_types=[
            pltpu.VMEM((CHUNK,), jnp.int32),
            pltpu.VMEM((CHUNK, D), jnp.float32),
            pltpu.SemaphoreType.DMA,
        ],
        compiler_params=pltpu.CompilerParams(use_tc_tiling_on_sc=False),
    )
    def emb(x_hbm, table_hbm, out_hbm, idx_v, rows_v, sem):
        wid = lax.axis_index("s") * 2 + lax.axis_index("c")
        base = wid * b_per_w

        def chunk_body(i):
            off = base + i * CHUNK
            pltpu.sync_copy(x_hbm.at[pl.ds(off, CHUNK)], idx_v)
            pltpu.async_copy(table_hbm.at[idx_v], rows_v, sem).wait()

            def row_body(r):
                for c in range(0, D, 16):
                    rows_v[r, pl.ds(c, 16)] = rows_v[r, pl.ds(c, 16)] * SCALE

            pl.loop(0, CHUNK)(row_body)
            pltpu.sync_copy(rows_v, out_hbm.at[pl.ds(off, CHUNK)])

        pl.loop(0, n_chunks)(chunk_body)

    return emb


def kernel(x, table):
    b, s = x.shape
    x_flat = x.reshape(b * s).astype(jnp.int32)
    out = _make_emb_kernel(b * s)(x_flat, table)
    return out.reshape(b, s, D)


# speedup vs baseline: 1.2542x; 1.2542x over previous
import functools
import math

import jax
import jax.numpy as jnp
from jax import lax
from jax.experimental import pallas as pl
from jax.experimental.pallas import tpu as pltpu
from jax.experimental.pallas import tpu_sc as plsc

D = 64
NW = 32
CHUNK = 128
NBUF = 2
SCALE = 1.0 / math.sqrt(D)


def _make_emb_kernel(b_total: int):
    b_per_w = b_total // NW
    n_chunks = b_per_w // CHUNK
    mesh = plsc.VectorSubcoreMesh(core_axis_name="c", subcore_axis_name="s")

    @functools.partial(
        pl.kernel,
        out_type=jax.ShapeDtypeStruct((b_total, D), jnp.float32),
        mesh=mesh,
        scratch_types=[
            pltpu.VMEM((n_chunks, CHUNK), jnp.int32),
            [pltpu.VMEM((CHUNK, D), jnp.float32) for _ in range(NBUF)],
            [pltpu.VMEM((CHUNK, D), jnp.float32) for _ in range(NBUF)],
            [pltpu.SemaphoreType.DMA for _ in range(NBUF)],
            [pltpu.SemaphoreType.DMA for _ in range(NBUF)],
        ],
        compiler_params=pltpu.CompilerParams(use_tc_tiling_on_sc=False),
    )
    def emb(x_hbm, table_hbm, out_hbm, idx_all, rows, srows, gsems, osems):
        wid = lax.axis_index("s") * 2 + lax.axis_index("c")
        base = wid * b_per_w

        pltpu.sync_copy(x_hbm.at[wid], idx_all)

        for b in range(NBUF):
            pltpu.async_copy(table_hbm.at[idx_all.at[b]], rows[b], gsems[b])

        def chunk_pair(i0):
            for b in range(NBUF):
                i = i0 + b
                pltpu.make_async_copy(
                    table_hbm.at[idx_all.at[i]], rows[b], gsems[b]
                ).wait()

                @pl.when(i >= NBUF)
                def _():
                    pltpu.make_async_copy(
                        srows[b], out_hbm.at[pl.ds((i - NBUF) * CHUNK, CHUNK)],
                        osems[b],
                    ).wait()

                def scale_row(r):
                    for c in range(0, D, 16):
                        srows[b][r, pl.ds(c, 16)] = (
                            rows[b][r, pl.ds(c, 16)] * SCALE
                        )

                plsc.parallel_loop(0, CHUNK, unroll=4)(scale_row)

                @pl.when(i + NBUF < n_chunks)
                def _():
                    pltpu.async_copy(
                        table_hbm.at[idx_all.at[i + NBUF]], rows[b], gsems[b]
                    )

                pltpu.async_copy(
                    srows[b], out_hbm.at[pl.ds(base + i * CHUNK, CHUNK)],
                    osems[b],
                )

        pl.loop(0, n_chunks, step=NBUF)(chunk_pair)

        for b in range(NBUF):
            pltpu.make_async_copy(
                srows[b], out_hbm.at[pl.ds(0, CHUNK)], osems[b]
            ).wait()

    return emb


def kernel(x, table):
    b, s = x.shape
    b_total = b * s
    x_grouped = x.reshape(NW, (b_total // NW) // CHUNK, CHUNK).astype(jnp.int32)
    out = _make_emb_kernel(b_total)(x_grouped, table)
    return out.reshape(b, s, D)
